# baseline (speedup 1.0000x reference)
import functools

import jax
import jax.numpy as jnp
from jax import lax
from jax.experimental import pallas as pl
from jax.experimental.pallas import tpu as pltpu
from jax.experimental.pallas import tpu_sc as plsc

L = 16


@functools.lru_cache(maxsize=None)
def _make_bmu_kernel(B: int, N: int, D: int):
    nchunk = N // L
    mesh = plsc.VectorSubcoreMesh(
        core_axis_name="c", subcore_axis_name="s", num_cores=1)

    @functools.partial(
        pl.kernel,
        mesh=mesh,
        out_type=jax.ShapeDtypeStruct((B * D,), jnp.float32),
        compiler_params=pltpu.CompilerParams(
            needs_layout_passes=False,
            skip_device_barrier=True,
            disable_bounds_checks=True,
            disable_semaphore_checks=True,
        ),
        scratch_types=[
            pltpu.VMEM((D,), jnp.float32),
            pltpu.VMEM((D * N,), jnp.float32),
            pltpu.VMEM((D,), jnp.float32),
        ],
    )
    def bmu(x_hbm, wt_hbm, out_hbm, x_v, wt_v, row_v):
        s = lax.axis_index("s")

        @pl.when(s < B)
        def _():
            pltpu.sync_copy(x_hbm.at[pl.ds(s * D, D)], x_v)
            pltpu.sync_copy(wt_hbm, wt_v)

            iota = lax.iota(jnp.int32, L)

            def body(d, accs):
                xd = plsc.load_gather(x_v, [jnp.full((L,), d, jnp.int32)])
                new = []
                for cc in range(nchunk):
                    wv = wt_v[pl.ds(d * N + cc * L, L)]
                    df = xd - wv
                    new.append(accs[cc] + df * df)
                return tuple(new)

            accs = lax.fori_loop(
                0, D, body,
                tuple(jnp.zeros((L,), jnp.float32) for _ in range(nchunk)))

            best_val = accs[0]
            best_idx = iota
            for cc in range(1, nchunk):
                m = accs[cc] < best_val
                best_val = jnp.where(m, accs[cc], best_val)
                best_idx = jnp.where(m, cc * L + iota, best_idx)
            mval = jnp.min(best_val)
            cand = jnp.where(best_val == mval, best_idx, jnp.int32(2**30))
            j = jnp.min(cand)

            for k in range(D // L):
                row_v[pl.ds(k * L, L)] = plsc.load_gather(
                    wt_v, [(iota + k * L) * N + j])
            pltpu.sync_copy(row_v, out_hbm.at[pl.ds(s * D, D)])

    return bmu


def kernel(x, weights):
    b, _, f = x.shape
    n = weights.shape[0]
    xq = x[:, -1, :].reshape(b * f)
    wt = weights.T.reshape(f * n)
    out = _make_bmu_kernel(b, n, f)(xq, wt)
    return out.reshape(b, f)


# speedup vs baseline: 1.0331x; 1.0331x over previous
import functools

import jax
import jax.numpy as jnp
from jax import lax
from jax.experimental import pallas as pl
from jax.experimental.pallas import tpu as pltpu
from jax.experimental.pallas import tpu_sc as plsc

L = 16
NH = 2
NS = 16


@functools.lru_cache(maxsize=None)
def _make_bmu_kernel(B: int, N: int, D: int):
    n_per_w = N // NH
    nchunk = n_per_w // L
    mesh = plsc.VectorSubcoreMesh(
        core_axis_name="c", subcore_axis_name="s", num_cores=1)

    @functools.partial(
        pl.kernel,
        mesh=mesh,
        out_type=jax.ShapeDtypeStruct((B * D,), jnp.float32),
        compiler_params=pltpu.CompilerParams(needs_layout_passes=False),
        scratch_types=[
            pltpu.VMEM((D,), jnp.float32),
            pltpu.VMEM((D * n_per_w,), jnp.float32),
            pltpu.VMEM((L,), jnp.float32),
            pltpu.VMEM((NH * L,), jnp.float32),
            pltpu.VMEM((D,), jnp.float32),
            pltpu.VMEM_SHARED((NS * L,), jnp.float32),
        ],
    )
    def bmu(x_hbm, wh_hbm, out_hbm, x_v, w_v, pub_v, mins_v, row_v, board):
        s = lax.axis_index("s")
        b = s // NH
        h = s % NH

        pltpu.sync_copy(x_hbm.at[pl.ds(b * D, D)], x_v)
        pltpu.sync_copy(wh_hbm.at[pl.ds(h * D * n_per_w, D * n_per_w)], w_v)

        iota = lax.iota(jnp.int32, L)

        def body(d, accs):
            xd = plsc.load_gather(x_v, [jnp.full((L,), d, jnp.int32)])
            new = []
            for cc in range(nchunk):
                wv = w_v[pl.ds(d * n_per_w + cc * L, L)]
                df = xd - wv
                new.append(accs[cc] + df * df)
            return tuple(new)

        accs = lax.fori_loop(
            0, D, body,
            tuple(jnp.zeros((L,), jnp.float32) for _ in range(nchunk)))

        best_val = accs[0]
        best_idx = iota
        for cc in range(1, nchunk):
            m = accs[cc] < best_val
            best_val = jnp.where(m, accs[cc], best_val)
            best_idx = jnp.where(m, cc * L + iota, best_idx)
        mval = jnp.min(best_val)
        cand = jnp.where(best_val == mval, best_idx, jnp.int32(2**30))
        j_loc = jnp.min(cand)

        pub_v[...] = jnp.full((L,), mval, jnp.float32)
        pltpu.sync_copy(pub_v, board.at[pl.ds(s * L, L)])
        plsc.subcore_barrier()

        pltpu.sync_copy(board.at[pl.ds((s // NH) * NH * L, NH * L)], mins_v)
        m0 = jnp.min(mins_v[pl.ds(0 * L, L)])
        m1 = jnp.min(mins_v[pl.ds(1 * L, L)])
        gmin = jnp.minimum(m0, m1)
        first_h = jnp.where(m0 == gmin, 0, 1)

        @pl.when(h == first_h)
        def _():
            for k in range(D // L):
                row_v[pl.ds(k * L, L)] = plsc.load_gather(
                    w_v, [(iota + k * L) * n_per_w + j_loc])
            pltpu.sync_copy(row_v, out_hbm.at[pl.ds(b * D, D)])

    return bmu


def kernel(x, weights):
    b, _, f = x.shape
    n = weights.shape[0]
    xq = x[:, -1, :].reshape(b * f)
    wh = weights.T.reshape(f, NH, n // NH).transpose(1, 0, 2).reshape(-1)
    out = _make_bmu_kernel(b, n, f)(xq, wh)
    return out.reshape(b, f)


# speedup vs baseline: 1.0625x; 1.0285x over previous
import functools

import jax
import jax.numpy as jnp
from jax import lax
from jax.experimental import pallas as pl
from jax.experimental.pallas import tpu as pltpu
from jax.experimental.pallas import tpu_sc as plsc

L = 16
NH = 2
NS = 16


@functools.lru_cache(maxsize=None)
def _make_bmu_kernel(B: int, N: int, D: int):
    n_per_w = N // NH
    nchunk = n_per_w // L
    mesh = plsc.VectorSubcoreMesh(
        core_axis_name="c", subcore_axis_name="s", num_cores=1)

    @functools.partial(
        pl.kernel,
        mesh=mesh,
        out_type=jax.ShapeDtypeStruct((B * D,), jnp.float32),
        compiler_params=pltpu.CompilerParams(needs_layout_passes=False),
        scratch_types=[
            pltpu.VMEM((D,), jnp.float32),
            pltpu.VMEM((D * n_per_w,), jnp.float32),
            pltpu.VMEM((L,), jnp.float32),
            pltpu.VMEM((NH * L,), jnp.float32),
            pltpu.VMEM((D,), jnp.float32),
            pltpu.VMEM_SHARED((NS * L,), jnp.float32),
            pltpu.SemaphoreType.DMA,
            pltpu.SemaphoreType.DMA,
            pltpu.SemaphoreType.DMA,
        ],
    )
    def bmu(x_hbm, wh_hbm, out_hbm, x_v, w_v, pub_v, mins_v, row_v, board,
            sem_x, sem_w1, sem_w2):
        s = lax.axis_index("s")
        b = s // NH
        h = s % NH

        half = D * n_per_w // 2
        woff = h * D * n_per_w
        cx = pltpu.async_copy(x_hbm.at[pl.ds(b * D, D)], x_v, sem_x)
        cw1 = pltpu.async_copy(
            wh_hbm.at[pl.ds(woff, half)], w_v.at[pl.ds(0, half)], sem_w1)
        cw2 = pltpu.async_copy(
            wh_hbm.at[pl.ds(woff + half, half)], w_v.at[pl.ds(half, half)],
            sem_w2)
        cx.wait()
        cw1.wait()

        iota = lax.iota(jnp.int32, L)

        def body(d, accs):
            xd = plsc.load_gather(x_v, [jnp.full((L,), d, jnp.int32)])
            new = []
            for cc in range(nchunk):
                wv = w_v[pl.ds(d * n_per_w + cc * L, L)]
                df = xd - wv
                new.append(accs[cc] + df * df)
            return tuple(new)

        accs = lax.fori_loop(
            0, D // 2, body,
            tuple(jnp.zeros((L,), jnp.float32) for _ in range(nchunk)))
        cw2.wait()
        accs = lax.fori_loop(D // 2, D, body, accs)

        best_val = accs[0]
        best_idx = iota
        for cc in range(1, nchunk):
            m = accs[cc] < best_val
            best_val = jnp.where(m, accs[cc], best_val)
            best_idx = jnp.where(m, cc * L + iota, best_idx)
        mval = jnp.min(best_val)
        cand = jnp.where(best_val == mval, best_idx, jnp.int32(2**30))
        j_loc = jnp.min(cand)

        pub_v[...] = jnp.full((L,), mval, jnp.float32)
        pltpu.sync_copy(pub_v, board.at[pl.ds(s * L, L)])
        plsc.subcore_barrier()

        pltpu.sync_copy(board.at[pl.ds((s // NH) * NH * L, NH * L)], mins_v)
        m0 = jnp.min(mins_v[pl.ds(0 * L, L)])
        m1 = jnp.min(mins_v[pl.ds(1 * L, L)])
        gmin = jnp.minimum(m0, m1)
        first_h = jnp.where(m0 == gmin, 0, 1)

        @pl.when(h == first_h)
        def _():
            for k in range(D // L):
                row_v[pl.ds(k * L, L)] = plsc.load_gather(
                    w_v, [(iota + k * L) * n_per_w + j_loc])
            pltpu.sync_copy(row_v, out_hbm.at[pl.ds(b * D, D)])

    return bmu


def kernel(x, weights):
    b, _, f = x.shape
    n = weights.shape[0]
    xq = x[:, -1, :].reshape(b * f)
    wh = weights.T.reshape(f, NH, n // NH).transpose(1, 0, 2).reshape(-1)
    out = _make_bmu_kernel(b, n, f)(xq, wh)
    return out.reshape(b, f)


# speedup vs baseline: 1.0647x; 1.0020x over previous
import functools

import jax
import jax.numpy as jnp
from jax import lax
from jax.experimental import pallas as pl
from jax.experimental.pallas import tpu as pltpu
from jax.experimental.pallas import tpu_sc as plsc

L = 16
NH = 2
NS = 16


@functools.lru_cache(maxsize=None)
def _make_bmu_kernel(B: int, N: int, D: int):
    n_per_w = N // NH
    nchunk = n_per_w // L
    mesh = plsc.VectorSubcoreMesh(
        core_axis_name="c", subcore_axis_name="s", num_cores=1)

    @functools.partial(
        pl.kernel,
        mesh=mesh,
        out_type=jax.ShapeDtypeStruct((B * D,), jnp.float32),
        compiler_params=pltpu.CompilerParams(needs_layout_passes=False),
        scratch_types=[
            pltpu.VMEM((D,), jnp.float32),
            pltpu.VMEM((D * n_per_w,), jnp.float32),
            pltpu.VMEM((L,), jnp.float32),
            pltpu.VMEM((NH * L,), jnp.float32),
            pltpu.VMEM((D,), jnp.float32),
            pltpu.VMEM_SHARED((NS * L,), jnp.float32),
            pltpu.SemaphoreType.DMA,
            pltpu.SemaphoreType.DMA,
            pltpu.SemaphoreType.DMA,
            pltpu.SemaphoreType.DMA,
            pltpu.SemaphoreType.DMA,
        ],
    )
    def bmu(x_hbm, wh_hbm, out_hbm, x_v, w_v, pub_v, mins_v, row_v, board,
            sem_x, sem_w1, sem_w2, sem_w3, sem_w4):
        s = lax.axis_index("s")
        b = s // NH
        h = s % NH

        nst = 4
        chunk = D * n_per_w // nst
        woff = h * D * n_per_w
        cx = pltpu.async_copy(x_hbm.at[pl.ds(b * D, D)], x_v, sem_x)
        sems = [sem_w1, sem_w2, sem_w3, sem_w4]
        copies = [
            pltpu.async_copy(
                wh_hbm.at[pl.ds(woff + k * chunk, chunk)],
                w_v.at[pl.ds(k * chunk, chunk)], sems[k])
            for k in range(nst)
        ]
        cx.wait()

        iota = lax.iota(jnp.int32, L)

        def body(d, accs):
            xd = plsc.load_gather(x_v, [jnp.full((L,), d, jnp.int32)])
            new = []
            for cc in range(nchunk):
                wv = w_v[pl.ds(d * n_per_w + cc * L, L)]
                df = xd - wv
                new.append(accs[cc] + df * df)
            return tuple(new)

        accs = tuple(jnp.zeros((L,), jnp.float32) for _ in range(nchunk))
        dstep = D // nst
        for k in range(nst):
            copies[k].wait()
            accs = lax.fori_loop(k * dstep, (k + 1) * dstep, body, accs)

        best_val = accs[0]
        best_idx = iota
        for cc in range(1, nchunk):
            m = accs[cc] < best_val
            best_val = jnp.where(m, accs[cc], best_val)
            best_idx = jnp.where(m, cc * L + iota, best_idx)
        mval = jnp.min(best_val)
        cand = jnp.where(best_val == mval, best_idx, jnp.int32(2**30))
        j_loc = jnp.min(cand)

        pub_v[...] = jnp.full((L,), mval, jnp.float32)
        pltpu.sync_copy(pub_v, board.at[pl.ds(s * L, L)])
        plsc.subcore_barrier()

        pltpu.sync_copy(board.at[pl.ds((s // NH) * NH * L, NH * L)], mins_v)
        m0 = jnp.min(mins_v[pl.ds(0 * L, L)])
        m1 = jnp.min(mins_v[pl.ds(1 * L, L)])
        gmin = jnp.minimum(m0, m1)
        first_h = jnp.where(m0 == gmin, 0, 1)

        @pl.when(h == first_h)
        def _():
            for k in range(D // L):
                row_v[pl.ds(k * L, L)] = plsc.load_gather(
                    w_v, [(iota + k * L) * n_per_w + j_loc])
            pltpu.sync_copy(row_v, out_hbm.at[pl.ds(b * D, D)])

    return bmu


def kernel(x, weights):
    b, _, f = x.shape
    n = weights.shape[0]
    xq = x[:, -1, :].reshape(b * f)
    wh = weights.T.reshape(f, NH, n // NH).transpose(1, 0, 2).reshape(-1)
    out = _make_bmu_kernel(b, n, f)(xq, wh)
    return out.reshape(b, f)
